# baseline (speedup 1.0000x reference)
import functools

import jax
import jax.numpy as jnp
from jax import lax
from jax.experimental import pallas as pl
from jax.experimental.pallas import tpu as pltpu
from jax.experimental.pallas import tpu_sc as plsc

EMB = 128
NC, NS = 2, 16
NW = NC * NS
CH = 80
GROUPS = CH // 16


def _table_body(w0_ref, w1_ref, w2_ref, t_ref):
    c = lax.broadcasted_iota(jnp.int32, (64, 1), 0)
    i0, r = c // 12, c % 12
    i1, i2 = r // 2, r % 2

    def oh(idx):
        return (idx == lax.broadcasted_iota(jnp.int32, (1, 8), 1)
                ).astype(jnp.float32)

    t_ref[...] = (
        jnp.dot(oh(i0), w0_ref[...], preferred_element_type=jnp.float32)
        + jnp.dot(oh(i1), w1_ref[...], preferred_element_type=jnp.float32)
        + jnp.dot(oh(i2), w2_ref[...], preferred_element_type=jnp.float32))


def _build_table(W0, W1, W2):
    def pad8(w):
        return jnp.zeros((8, EMB), jnp.float32).at[:w.shape[0]].set(w)

    return pl.pallas_call(
        _table_body,
        out_shape=jax.ShapeDtypeStruct((64, EMB), jnp.float32),
    )(pad8(W0), pad8(W1), pad8(W2))


def _make_sc_kernel(E):
    per_w = E // NW
    iters = per_w // CH
    mesh = plsc.VectorSubcoreMesh(core_axis_name="c", subcore_axis_name="s")

    @functools.partial(
        pl.kernel, mesh=mesh,
        out_type=jax.ShapeDtypeStruct((E, EMB), jnp.float32),
        scratch_types=[
            pltpu.VMEM((3, CH), jnp.int32),
            pltpu.VMEM((CH,), jnp.int32),
            pltpu.VMEM((CH, EMB), jnp.float32),
            pltpu.SemaphoreType.DMA,
        ])
    def k(attr_hbm, t_hbm, out_hbm, attr_v, codes_v, rows_v, sem):
        wid = lax.axis_index("s") * NC + lax.axis_index("c")
        tile_base = wid * per_w

        def body(i, carry):
            base = tile_base + i * CH
            for c in range(3):
                pltpu.sync_copy(attr_hbm.at[pl.ds(c * E + base, CH)],
                                attr_v.at[c])
            for j in range(GROUPS):
                s = pl.ds(j * 16, 16)
                codes_v[s] = (attr_v[0, s] * 12 + attr_v[1, s] * 2
                              + attr_v[2, s])
            pltpu.async_copy(t_hbm.at[codes_v], rows_v, sem).wait()
            pltpu.sync_copy(rows_v, out_hbm.at[pl.ds(base, CH)])
            return carry

        lax.fori_loop(0, iters, body, 0)

    return k


def kernel(edge_attr, W0, W1, W2):
    E = edge_attr.shape[0]
    attr = edge_attr.astype(jnp.int32).T.reshape(-1)
    t = _build_table(W0, W1, W2)
    return _make_sc_kernel(E)(attr, t)


# speedup vs baseline: 1.0093x; 1.0093x over previous
import functools

import jax
import jax.numpy as jnp
from jax import lax
from jax.experimental import pallas as pl
from jax.experimental.pallas import tpu as pltpu
from jax.experimental.pallas import tpu_sc as plsc

EMB = 128
NC, NS = 2, 16
NW = NC * NS
CH = 80
GROUPS = CH // 16


def _table_body(w0_ref, w1_ref, w2_ref, t_ref):
    c = lax.broadcasted_iota(jnp.int32, (64, 1), 0)
    i0, r = c // 12, c % 12
    i1, i2 = r // 2, r % 2

    def oh(idx):
        return (idx == lax.broadcasted_iota(jnp.int32, (1, 8), 1)
                ).astype(jnp.float32)

    t_ref[...] = (
        jnp.dot(oh(i0), w0_ref[...], preferred_element_type=jnp.float32)
        + jnp.dot(oh(i1), w1_ref[...], preferred_element_type=jnp.float32)
        + jnp.dot(oh(i2), w2_ref[...], preferred_element_type=jnp.float32))


def _build_table(W0, W1, W2):
    def pad8(w):
        return jnp.zeros((8, EMB), jnp.float32).at[:w.shape[0]].set(w)

    return pl.pallas_call(
        _table_body,
        out_shape=jax.ShapeDtypeStruct((64, EMB), jnp.float32),
    )(pad8(W0), pad8(W1), pad8(W2))


def _make_sc_kernel(E):
    per_w = E // NW
    chunk = 200
    sub = 40
    nsub = chunk // sub
    groups16 = per_w // 16
    n_groups = per_w // (2 * chunk)
    mesh = plsc.VectorSubcoreMesh(core_axis_name="c", subcore_axis_name="s")

    @functools.partial(
        pl.kernel, mesh=mesh,
        out_type=jax.ShapeDtypeStruct((E, EMB), jnp.float32),
        scratch_types=[
            pltpu.VMEM((per_w,), jnp.int32),
            pltpu.VMEM((per_w,), jnp.int32),
            pltpu.VMEM((per_w,), jnp.int32),
            pltpu.VMEM((per_w,), jnp.int32),
            pltpu.VMEM((chunk, EMB), jnp.float32),
            pltpu.VMEM((chunk, EMB), jnp.float32),
            pltpu.SemaphoreType.DMA,
            pltpu.SemaphoreType.DMA,
            pltpu.SemaphoreType.DMA,
        ])
    def k(attr_hbm, t_hbm, out_hbm, a0_v, a1_v, a2_v, codes_v, rows0, rows1,
          sem_g, sem_o0, sem_o1):
        wid = lax.axis_index("s") * NC + lax.axis_index("c")
        tile_base = wid * per_w

        for c, av in enumerate((a0_v, a1_v, a2_v)):
            pltpu.sync_copy(attr_hbm.at[pl.ds(c * E + tile_base, per_w)], av)

        def code_body(j, carry):
            s = pl.ds(j * 16, 16)
            codes_v[s] = a0_v[s] * 12 + a1_v[s] * 2 + a2_v[s]
            return carry

        lax.fori_loop(0, groups16, code_body, 0, unroll=5)

        rows = (rows0, rows1)
        sems_o = (sem_o0, sem_o1)

        def chunk_body(g, carry):
            for b in range(2):
                base = (2 * g + b) * chunk
                rb, so = rows[b], sems_o[b]

                @pl.when(g >= 1)
                def _():
                    for j in range(nsub):
                        pltpu.make_async_copy(
                            rb.at[pl.ds(j * sub, sub)],
                            out_hbm.at[pl.ds(tile_base + base + j * sub, sub)],
                            so).wait()

                hs = [pltpu.async_copy(
                          t_hbm.at[codes_v.at[pl.ds(base + j * sub, sub)]],
                          rb.at[pl.ds(j * sub, sub)], sem_g)
                      for j in range(nsub)]
                for h in hs:
                    h.wait()
                for j in range(nsub):
                    pltpu.async_copy(
                        rb.at[pl.ds(j * sub, sub)],
                        out_hbm.at[pl.ds(tile_base + base + j * sub, sub)],
                        so)
            return carry

        lax.fori_loop(0, n_groups, chunk_body, 0)

        for b in range(2):
            base = (2 * (n_groups - 1) + b) * chunk
            for j in range(nsub):
                pltpu.make_async_copy(
                    rows[b].at[pl.ds(j * sub, sub)],
                    out_hbm.at[pl.ds(tile_base + base + j * sub, sub)],
                    sems_o[b]).wait()

    return k


def kernel(edge_attr, W0, W1, W2):
    E = edge_attr.shape[0]
    attr = edge_attr.astype(jnp.int32).T.reshape(-1)
    t = _build_table(W0, W1, W2)
    return _make_sc_kernel(E)(attr, t)


# speedup vs baseline: 5.0082x; 4.9622x over previous
import functools

import jax
import jax.numpy as jnp
from jax import lax
from jax.experimental import pallas as pl
from jax.experimental.pallas import tpu as pltpu
from jax.experimental.pallas import tpu_sc as plsc

EMB = 128
NC, NS = 2, 16
NW = NC * NS
CH = 80
GROUPS = CH // 16


def _table_body(w0_ref, w1_ref, w2_ref, t_ref):
    c = lax.broadcasted_iota(jnp.int32, (64, 1), 0)
    i0, r = c // 12, c % 12
    i1, i2 = r // 2, r % 2

    def oh(idx):
        return (idx == lax.broadcasted_iota(jnp.int32, (1, 8), 1)
                ).astype(jnp.float32)

    t_ref[...] = (
        jnp.dot(oh(i0), w0_ref[...], preferred_element_type=jnp.float32)
        + jnp.dot(oh(i1), w1_ref[...], preferred_element_type=jnp.float32)
        + jnp.dot(oh(i2), w2_ref[...], preferred_element_type=jnp.float32))


def _build_table(W0, W1, W2):
    def pad8(w):
        return jnp.zeros((8, EMB), jnp.float32).at[:w.shape[0]].set(w)

    t = pl.pallas_call(
        _table_body,
        out_shape=jax.ShapeDtypeStruct((64, EMB), jnp.float32),
    )(pad8(W0), pad8(W1), pad8(W2))
    return jnp.broadcast_to(t, (NW, 64, EMB)).reshape(NW * 64, EMB)


def _make_sc_kernel(E):
    per_w = E // NW
    chunk = 200
    sub = 40
    nsub = chunk // sub
    groups16 = per_w // 16
    n_groups = per_w // (2 * chunk)
    mesh = plsc.VectorSubcoreMesh(core_axis_name="c", subcore_axis_name="s")

    @functools.partial(
        pl.kernel, mesh=mesh,
        out_type=jax.ShapeDtypeStruct((E, EMB), jnp.float32),
        scratch_types=[
            pltpu.VMEM((per_w,), jnp.int32),
            pltpu.VMEM((per_w,), jnp.int32),
            pltpu.VMEM((per_w,), jnp.int32),
            pltpu.VMEM((per_w,), jnp.int32),
            pltpu.VMEM((chunk, EMB), jnp.float32),
            pltpu.VMEM((chunk, EMB), jnp.float32),
            pltpu.SemaphoreType.DMA,
            pltpu.SemaphoreType.DMA,
            pltpu.SemaphoreType.DMA,
        ])
    def k(attr_hbm, t_hbm, out_hbm, a0_v, a1_v, a2_v, codes_v, rows0, rows1,
          sem_g, sem_o0, sem_o1):
        wid = lax.axis_index("s") * NC + lax.axis_index("c")
        tile_base = wid * per_w

        for c, av in enumerate((a0_v, a1_v, a2_v)):
            pltpu.sync_copy(attr_hbm.at[pl.ds(c * E + tile_base, per_w)], av)

        code_off = wid * 64

        def code_body(j, carry):
            s = pl.ds(j * 16, 16)
            codes_v[s] = a0_v[s] * 12 + a1_v[s] * 2 + a2_v[s] + code_off
            return carry

        lax.fori_loop(0, groups16, code_body, 0, unroll=5)

        rows = (rows0, rows1)
        sems_o = (sem_o0, sem_o1)

        def chunk_body(g, carry):
            for b in range(2):
                base = (2 * g + b) * chunk
                rb, so = rows[b], sems_o[b]

                @pl.when(g >= 1)
                def _():
                    for j in range(nsub):
                        pltpu.make_async_copy(
                            rb.at[pl.ds(j * sub, sub)],
                            out_hbm.at[pl.ds(tile_base + base + j * sub, sub)],
                            so).wait()

                hs = [pltpu.async_copy(
                          t_hbm.at[codes_v.at[pl.ds(base + j * sub, sub)]],
                          rb.at[pl.ds(j * sub, sub)], sem_g)
                      for j in range(nsub)]
                for h in hs:
                    h.wait()
                for j in range(nsub):
                    pltpu.async_copy(
                        rb.at[pl.ds(j * sub, sub)],
                        out_hbm.at[pl.ds(tile_base + base + j * sub, sub)],
                        so)
            return carry

        lax.fori_loop(0, n_groups, chunk_body, 0)

        for b in range(2):
            base = (2 * (n_groups - 1) + b) * chunk
            for j in range(nsub):
                pltpu.make_async_copy(
                    rows[b].at[pl.ds(j * sub, sub)],
                    out_hbm.at[pl.ds(tile_base + base + j * sub, sub)],
                    sems_o[b]).wait()

    return k


def kernel(edge_attr, W0, W1, W2):
    E = edge_attr.shape[0]
    attr = edge_attr.astype(jnp.int32).T.reshape(-1)
    t = _build_table(W0, W1, W2)
    return _make_sc_kernel(E)(attr, t)
